# baseline (speedup 1.0000x reference)
import jax
import jax.numpy as jnp
from jax import lax
from jax.experimental import pallas as pl
from jax.experimental.pallas import tpu as pltpu
from jax.experimental.pallas import tpu_sc as plsc

BATCH = 4096
SEQ = 200
DIM = 64
NC, NS, L = 2, 16, 16
NW = NC * NS
ROWS_W = BATCH // NW
OUT_S = SEQ - 1
HALF = 2 * OUT_S
IDX_W = ROWS_W * SEQ
NVEC = (HALF + L - 1) // L

_CHUNKS = ((0, 128), (128, 128), (256, 128), (384, HALF - 384))


def _body(ws_hbm, tab_hbm, out_hbm, idx_v, idx2_v, rows_v, sem):
    wid = lax.axis_index("s") * NC + lax.axis_index("c")
    pltpu.sync_copy(ws_hbm.at[pl.ds(wid * IDX_W, IDX_W)], idx_v.at[pl.ds(0, IDX_W)])
    j = lax.iota(jnp.int32, L)
    pat = ((j >> 1) + (j & 1)).reshape(L, 1)
    dnums = lax.GatherDimensionNumbers(
        offset_dims=(), collapsed_slice_dims=(0,), start_index_map=(0,)
    )

    def step(i, carry):
        base = i * SEQ
        for k in range(NVEC):
            window = idx_v[pl.ds(base + 8 * k, L)]
            idx2_v[pl.ds(L * k, L)] = lax.gather(
                window,
                pat,
                dnums,
                slice_sizes=(1,),
                mode=lax.GatherScatterMode.PROMISE_IN_BOUNDS,
            )
        copies = [
            pltpu.async_copy(
                tab_hbm.at[idx2_v.at[pl.ds(off, n)]],
                rows_v.at[pl.ds(off, n)],
                sem,
            )
            for off, n in _CHUNKS
        ]
        for c in copies:
            c.wait()
        pltpu.sync_copy(rows_v, out_hbm.at[wid * ROWS_W + i])
        return carry

    lax.fori_loop(0, ROWS_W, step, 0)


@jax.jit
def kernel(whitespace, embed_table):
    ws_flat = whitespace.reshape(-1).astype(jnp.int32)
    mesh = plsc.VectorSubcoreMesh(
        core_axis_name="c", subcore_axis_name="s", num_cores=NC, num_subcores=NS
    )
    out = pl.kernel(
        _body,
        out_type=jax.ShapeDtypeStruct((BATCH, 2 * SEQ, DIM), jnp.float32),
        mesh=mesh,
        compiler_params=pltpu.CompilerParams(use_tc_tiling_on_sc=False),
        scratch_types=[
            pltpu.VMEM((IDX_W + 2 * L,), jnp.int32),
            pltpu.VMEM((NVEC * L,), jnp.int32),
            pltpu.VMEM((2 * SEQ, DIM), jnp.float32),
            pltpu.SemaphoreType.DMA,
        ],
    )(ws_flat, embed_table)
    return out.reshape(BATCH, SEQ, 2 * DIM)[:, :OUT_S, :]


# speedup vs baseline: 1.9512x; 1.9512x over previous
import jax
import jax.numpy as jnp
from jax import lax
from jax.experimental import pallas as pl
from jax.experimental.pallas import tpu as pltpu
from jax.experimental.pallas import tpu_sc as plsc

BATCH = 4096
SEQ = 200
DIM = 64
NC, NS, L = 2, 16, 16
NW = NC * NS
ROWS_W = BATCH // NW
OUT_S = SEQ - 1
IDX_W = ROWS_W * SEQ

_CHUNKS = ((0, 128), (128, SEQ - 128))


def _body(ws_hbm, tab_hbm, out_hbm, idxl_v, idxr_v, left_v, right_v, sem):
    wid = lax.axis_index("s") * NC + lax.axis_index("c")
    pltpu.sync_copy(ws_hbm.at[pl.ds(wid * IDX_W, IDX_W)], idxl_v.at[pl.ds(0, IDX_W)])
    zeros = lax.iota(jnp.int32, L) * 0
    idxl_v[pl.ds(IDX_W, L)] = zeros

    def shift(k, carry):
        idxr_v[pl.ds(k * L, L)] = idxl_v[pl.ds(k * L + 1, L)]
        return carry

    lax.fori_loop(0, IDX_W // L, shift, 0)

    def step(i, carry):
        base = i * SEQ
        copies = []
        for off, n in _CHUNKS:
            copies.append(
                pltpu.async_copy(
                    tab_hbm.at[idxl_v.at[pl.ds(base + off, n)]],
                    left_v.at[pl.ds(off, n)],
                    sem,
                )
            )
            copies.append(
                pltpu.async_copy(
                    tab_hbm.at[idxr_v.at[pl.ds(base + off, n)]],
                    right_v.at[pl.ds(off, n)],
                    sem,
                )
            )
        for c in copies:
            c.wait()
        row0 = (wid * ROWS_W + i) * SEQ
        pltpu.sync_copy(left_v, out_hbm.at[pl.ds(row0, SEQ), pl.ds(0, DIM)])
        pltpu.sync_copy(right_v, out_hbm.at[pl.ds(row0, SEQ), pl.ds(DIM, DIM)])
        return carry

    lax.fori_loop(0, ROWS_W, step, 0)


@jax.jit
def kernel(whitespace, embed_table):
    ws_flat = whitespace.reshape(-1).astype(jnp.int32)
    mesh = plsc.VectorSubcoreMesh(
        core_axis_name="c", subcore_axis_name="s", num_cores=NC, num_subcores=NS
    )
    out = pl.kernel(
        _body,
        out_type=jax.ShapeDtypeStruct((BATCH * SEQ, 2 * DIM), jnp.float32),
        mesh=mesh,
        compiler_params=pltpu.CompilerParams(use_tc_tiling_on_sc=False),
        scratch_types=[
            pltpu.VMEM((IDX_W + L,), jnp.int32),
            pltpu.VMEM((IDX_W,), jnp.int32),
            pltpu.VMEM((SEQ, DIM), jnp.float32),
            pltpu.VMEM((SEQ, DIM), jnp.float32),
            pltpu.SemaphoreType.DMA,
        ],
    )(ws_flat, embed_table)
    return out.reshape(BATCH, SEQ, 2 * DIM)[:, :OUT_S, :]


# speedup vs baseline: 2.8823x; 1.4772x over previous
import jax
import jax.numpy as jnp
from jax import lax
from jax.experimental import pallas as pl
from jax.experimental.pallas import tpu as pltpu
from jax.experimental.pallas import tpu_sc as plsc

BATCH = 4096
SEQ = 200
DIM = 64
NC, NS, L = 2, 16, 16
NW = NC * NS
BS_W = BATCH // NW
OUT_S = SEQ - 1


def _body(ws_hbm, tab_hbm, out_hbm, idx_v, buf0_v, buf1_v, gsem, wsem0, wsem1):
    wid = lax.axis_index("s") * NC + lax.axis_index("c")
    b0 = wid * BS_W
    pltpu.sync_copy(ws_hbm.at[:, pl.ds(b0, BS_W)], idx_v)
    bufs = (buf0_v, buf1_v)
    wsems = (wsem0, wsem1)

    def drain(k, count):
        for _ in range(count):
            pltpu.make_async_copy(
                bufs[k], out_hbm.at[pl.ds(0, BS_W), pl.ds(0, DIM)], wsems[k]
            ).wait()

    def issue(c, k):
        pltpu.async_copy(tab_hbm.at[idx_v.at[c]], bufs[k], gsem).wait()

        @pl.when(c < OUT_S)
        def _():
            pltpu.async_copy(
                bufs[k],
                out_hbm.at[pl.ds(c * BATCH + b0, BS_W), pl.ds(0, DIM)],
                wsems[k],
            )

        @pl.when(c > 0)
        def _():
            pltpu.async_copy(
                bufs[k],
                out_hbm.at[pl.ds((c - 1) * BATCH + b0, BS_W), pl.ds(DIM, DIM)],
                wsems[k],
            )

    def step(t, carry):
        @pl.when(t == 1)
        def _():
            drain(0, 1)

        @pl.when(t >= 2)
        def _():
            drain(0, 2)

        issue(2 * t, 0)

        @pl.when(t >= 1)
        def _():
            drain(1, 2)

        issue(2 * t + 1, 1)
        return carry

    lax.fori_loop(0, SEQ // 2, step, 0)
    drain(0, 2)
    drain(1, 1)


@jax.jit
def kernel(whitespace, embed_table):
    ws_t = whitespace.T.astype(jnp.int32)
    mesh = plsc.VectorSubcoreMesh(
        core_axis_name="c", subcore_axis_name="s", num_cores=NC, num_subcores=NS
    )
    out = pl.kernel(
        _body,
        out_type=jax.ShapeDtypeStruct((OUT_S * BATCH, 2 * DIM), jnp.float32),
        mesh=mesh,
        compiler_params=pltpu.CompilerParams(use_tc_tiling_on_sc=False),
        scratch_types=[
            pltpu.VMEM((SEQ, BS_W), jnp.int32),
            pltpu.VMEM((BS_W, DIM), jnp.float32),
            pltpu.VMEM((BS_W, DIM), jnp.float32),
            pltpu.SemaphoreType.DMA,
            pltpu.SemaphoreType.DMA,
            pltpu.SemaphoreType.DMA,
        ],
    )(ws_t, embed_table)
    return out.reshape(OUT_S, BATCH, 2 * DIM).transpose(1, 0, 2)
